# Initial kernel scaffold; baseline (speedup 1.0000x reference)
import jax
import jax.numpy as jnp
from jax.experimental import pallas as pl


def kernel(x_bert, edge_index, batch_ids, W_b2f, b_b2f, gnn_W_self, gnn_W_nbr, gnn_b, mlp_W1, mlp_b1, mlp_W2, mlp_b2):
    raise NotImplementedError("write your pallas kernel here")


# speedup vs baseline: 12.4746x; 12.4746x over previous
import functools

import jax
import jax.numpy as jnp
from jax import lax
from jax.experimental import pallas as pl
from jax.experimental.pallas import tpu as pltpu
from jax.experimental.pallas import tpu_sc as plsc

_N = 10000
_E = 320000
_BERT = 768
_D = 128
_G = 64
_L = 2

_NC = 2
_NS = 16
_NW = _NC * _NS
_C = 120
_CH = 84
_EPT = _C * _CH
_E_PAD = _NW * _EPT
_ROWS_PER_TILE = 632
_N_PAD = _ROWS_PER_TILE * _NS


def _sc_edge_agg_body(x_hbm, src_hbm, dst_hbm, out_hbm,
                      sbuf0, sbuf1, sbuf2, dbuf0, dbuf1, dbuf2,
                      rows0_v, rows1_v, rows2_v, agg_sh,
                      semi0, semi1, semi2, sem0, sem1, sem2):
    cid = lax.axis_index("c")
    sid = lax.axis_index("s")
    wid = sid * _NC + cid
    ebase = wid * _EPT

    sbufs, dbufs, rows, semis, sems = ((sbuf0, sbuf1, sbuf2),
                                       (dbuf0, dbuf1, dbuf2),
                                       (rows0_v, rows1_v, rows2_v),
                                       (semi0, semi1, semi2),
                                       (sem0, sem1, sem2))

    def _idx_load(c, p):
        off = pl.multiple_of(ebase + c * _C, 8)
        pltpu.async_copy(src_hbm.at[pl.ds(off, _C)], sbufs[p], semis[p])
        pltpu.async_copy(dst_hbm.at[pl.ds(off, _C)], dbufs[p], semis[p])

    def _idx_wait(p):
        pltpu.make_async_copy(src_hbm.at[pl.ds(0, _C)], sbufs[p],
                              semis[p]).wait()
        pltpu.make_async_copy(dst_hbm.at[pl.ds(0, _C)], dbufs[p],
                              semis[p]).wait()

    def _gather(p):
        pltpu.async_copy(x_hbm.at[sbufs[p]], rows[p], sems[p])

    def _gather_wait(p):
        pltpu.make_async_copy(x_hbm.at[sbufs[p]], rows[p], sems[p]).wait()

    _idx_load(0, 0)

    zero = jnp.zeros((16,), jnp.float32)

    def _zero_row(i, _):
        for j in range(8):
            rows0_v[i, pl.ds(j * 16, 16)] = zero
        return _

    lax.fori_loop(0, _C, _zero_row, None, unroll=4)

    zbase = sid * _ROWS_PER_TILE
    nfull = _ROWS_PER_TILE // _C
    for k in range(nfull):
        pltpu.sync_copy(rows0_v, agg_sh.at[pl.ds(zbase + k * _C, _C)])
    rem = _ROWS_PER_TILE - nfull * _C
    pltpu.sync_copy(rows0_v.at[pl.ds(0, rem)],
                    agg_sh.at[pl.ds(zbase + nfull * _C, rem)])
    plsc.subcore_barrier()

    _idx_wait(0)
    _gather(0)
    _idx_load(1, 1)
    _idx_load(2, 2)

    def _chunk(c, q):
        _idx_wait((q + 1) % 3)
        _gather((q + 1) % 3)
        _gather_wait(q)
        pltpu.sync_copy(rows[q], agg_sh.at[dbufs[q]], add=True)
        _idx_load(jnp.minimum(c + 3, _CH - 1), q)

    def _six(k, _):
        for j in range(6):
            _chunk(6 * k + j, j % 3)
        return _

    lax.fori_loop(0, _CH // 6, _six, None)
    _gather_wait(0)
    _idx_wait(1)
    _idx_wait(2)
    plsc.subcore_barrier()

    pltpu.sync_copy(agg_sh.at[pl.ds(zbase, _ROWS_PER_TILE)],
                    out_hbm.at[cid, pl.ds(zbase, _ROWS_PER_TILE)])


@functools.partial(jax.jit, static_argnames=())
def _sc_edge_agg(x, src_p, dst_p):
    mesh = plsc.VectorSubcoreMesh(core_axis_name="c", subcore_axis_name="s")
    return pl.kernel(
        _sc_edge_agg_body,
        out_type=jax.ShapeDtypeStruct((_NC, _N_PAD, _D), jnp.float32),
        mesh=mesh,
        scratch_types=(
            [pltpu.VMEM((_C,), jnp.int32)] * 6
            + [pltpu.VMEM((_C, _D), jnp.float32)] * 3
            + [pltpu.VMEM_SHARED((_N_PAD, _D), jnp.float32)]
            + [pltpu.SemaphoreType.DMA] * 6
        ),
    )(x, src_p, dst_p)


_RB = 2000


def _proj_body(xb_ref, w_ref, b_ref, o_ref):
    o_ref[...] = (
        jnp.dot(xb_ref[...], w_ref[...], preferred_element_type=jnp.float32)
        + b_ref[...]
    )


def _proj(x_bert, W, b):
    return pl.pallas_call(
        _proj_body,
        grid=(_N // _RB,),
        in_specs=[
            pl.BlockSpec((_RB, _BERT), lambda i: (i, 0)),
            pl.BlockSpec((_BERT, _D), lambda i: (0, 0)),
            pl.BlockSpec((1, _D), lambda i: (0, 0)),
        ],
        out_specs=pl.BlockSpec((_RB, _D), lambda i: (i, 0)),
        out_shape=jax.ShapeDtypeStruct((_N, _D), jnp.float32),
    )(x_bert, W, b.reshape(1, _D))


def _layer_new_x(x_ref, p_ref, ws_ref, wn_ref, b_ref):
    x = x_ref[...]
    p = p_ref[...]
    agg = p[0] + p[1]
    v = (
        jnp.dot(x, ws_ref[...], preferred_element_type=jnp.float32)
        + jnp.dot(agg, wn_ref[...], preferred_element_type=jnp.float32)
        + b_ref[...]
    )
    return jnp.maximum(v, 0.01 * v) + x


def _layer_body(x_ref, p_ref, ws_ref, wn_ref, b_ref, o_ref):
    o_ref[...] = _layer_new_x(x_ref, p_ref, ws_ref, wn_ref, b_ref)


def _layer_update(x, parts, Ws, Wn, b):
    return pl.pallas_call(
        _layer_body,
        grid=(_N // _RB,),
        in_specs=[
            pl.BlockSpec((_RB, _D), lambda i: (i, 0)),
            pl.BlockSpec((_NC, _RB, _D), lambda i: (0, i, 0)),
            pl.BlockSpec((_D, _D), lambda i: (0, 0)),
            pl.BlockSpec((_D, _D), lambda i: (0, 0)),
            pl.BlockSpec((1, _D), lambda i: (0, 0)),
        ],
        out_specs=pl.BlockSpec((_RB, _D), lambda i: (i, 0)),
        out_shape=jax.ShapeDtypeStruct((_N, _D), jnp.float32),
    )(x, parts, Ws, Wn, b.reshape(1, _D))


def _fused_body(x_ref, p_ref, ws_ref, wn_ref, b_ref, bid_ref,
                w1_ref, b1_ref, w2_ref, b2_ref, o_ref,
                sums_scr, cnts_scr):
    i = pl.program_id(0)
    nsteps = pl.num_programs(0)

    @pl.when(i == 0)
    def _():
        sums_scr[...] = jnp.zeros_like(sums_scr)
        cnts_scr[...] = jnp.zeros_like(cnts_scr)

    xn = _layer_new_x(x_ref, p_ref, ws_ref, wn_ref, b_ref)
    bid = bid_ref[0]
    gids = lax.broadcasted_iota(jnp.int32, (_G, _RB), 0)
    mask = (bid == gids).astype(jnp.float32)
    sums_scr[...] += jnp.dot(mask, xn, preferred_element_type=jnp.float32)
    cnts_scr[...] += jnp.broadcast_to(
        jnp.sum(mask, axis=1, keepdims=True), (_G, _D))

    @pl.when(i == nsteps - 1)
    def _():
        pooled = sums_scr[...] / jnp.maximum(cnts_scr[...], 1.0)
        h = jnp.maximum(
            jnp.dot(pooled, w1_ref[...], preferred_element_type=jnp.float32)
            + b1_ref[...], 0.0)
        o_ref[...] = (
            jnp.dot(h, w2_ref[...], preferred_element_type=jnp.float32)
            + b2_ref[...]
        )


def _fused_layer_pool_mlp(x, parts, Ws, Wn, b, batch_ids, W1, b1, W2p, b2p):
    bid3 = batch_ids.reshape(_N // _RB, 1, _RB)
    return pl.pallas_call(
        _fused_body,
        grid=(_N // _RB,),
        in_specs=[
            pl.BlockSpec((_RB, _D), lambda i: (i, 0)),
            pl.BlockSpec((_NC, _RB, _D), lambda i: (0, i, 0)),
            pl.BlockSpec((_D, _D), lambda i: (0, 0)),
            pl.BlockSpec((_D, _D), lambda i: (0, 0)),
            pl.BlockSpec((1, _D), lambda i: (0, 0)),
            pl.BlockSpec((1, 1, _RB), lambda i: (i, 0, 0)),
            pl.BlockSpec((_D, _D), lambda i: (0, 0)),
            pl.BlockSpec((1, _D), lambda i: (0, 0)),
            pl.BlockSpec((_D, _D), lambda i: (0, 0)),
            pl.BlockSpec((1, _D), lambda i: (0, 0)),
        ],
        out_specs=pl.BlockSpec((_G, _D), lambda i: (0, 0)),
        out_shape=jax.ShapeDtypeStruct((_G, _D), jnp.float32),
        scratch_shapes=[
            pltpu.VMEM((_G, _D), jnp.float32),
            pltpu.VMEM((_G, _D), jnp.float32),
        ],
    )(x, parts, Ws, Wn, b.reshape(1, _D), bid3,
      W1, b1.reshape(1, _D), W2p, b2p.reshape(1, _D))


def kernel(x_bert, edge_index, batch_ids, W_b2f, b_b2f, gnn_W_self, gnn_W_nbr,
           gnn_b, mlp_W1, mlp_b1, mlp_W2, mlp_b2):
    src = edge_index[0].astype(jnp.int32)
    dst = edge_index[1].astype(jnp.int32)
    ppt = _EPT - _E // _NW
    pad_src = (jnp.arange(_NW * ppt, dtype=jnp.int32) % _N).reshape(_NW, ppt)
    pad_dst = (_N + jnp.arange(_NW * ppt, dtype=jnp.int32)
               % (_N_PAD - _N)).reshape(_NW, ppt)
    src_p = jnp.concatenate(
        [src.reshape(_NW, _E // _NW), pad_src], axis=1).reshape(-1)
    dst_p = jnp.concatenate(
        [dst.reshape(_NW, _E // _NW), pad_dst], axis=1).reshape(-1)

    x = _proj(x_bert, W_b2f, b_b2f)
    parts = _sc_edge_agg(x, src_p, dst_p)
    x = _layer_update(x, parts, gnn_W_self[0], gnn_W_nbr[0], gnn_b[0])
    parts = _sc_edge_agg(x, src_p, dst_p)

    W2p = jnp.pad(mlp_W2, ((0, 0), (0, _D - mlp_W2.shape[1])))
    b2p = jnp.pad(mlp_b2, (0, _D - mlp_b2.shape[0]))
    logits = _fused_layer_pool_mlp(
        x, parts, gnn_W_self[1], gnn_W_nbr[1], gnn_b[1],
        batch_ids.astype(jnp.int32), mlp_W1, mlp_b1, W2p, b2p)
    return logits[:, :mlp_W2.shape[1]]
